# baseline (speedup 1.0000x reference)
import jax
import jax.numpy as jnp
from jax import lax
from jax.experimental import pallas as pl
from jax.experimental.pallas import tpu as pltpu
from jax.experimental.pallas import tpu_sc as plsc

N = 10000
B = 4
U = 64
E0 = 160000
EP = 163840
CH = 128
EH = 5120
NH = EP // (16 * EH)
NP = 10240
NR = NP // 16
ETG = EP // 32
NT = 32


def _make_sc_state_pass():
    mesh = plsc.VectorSubcoreMesh(core_axis_name="c", subcore_axis_name="s")
    scratch = [
        pltpu.VMEM_SHARED((NP, 128), jnp.float32),
        pltpu.VMEM((EH,), jnp.int32),
        pltpu.VMEM((EH,), jnp.float32),
        pltpu.VMEM((EH // CH, CH), jnp.int32),
        pltpu.VMEM((CH, 128), jnp.float32),
        pltpu.VMEM((CH, 128), jnp.float32),
        pltpu.VMEM((NR,), jnp.float32),
        pltpu.SemaphoreType.DMA,
        pltpu.SemaphoreType.DMA,
    ]

    def body(xs, srch, dsth, wh, wlh,
             out128, acc, src_v, w_v, dst_v, rows0, rows1, wl_v,
             sem0, sem1):
        c = lax.axis_index("c")
        s = lax.axis_index("s")
        r0 = s * NR
        bufs = (rows0, rows1)
        sems = (sem0, sem1)

        def gather(j, p):
            return pltpu.make_async_copy(
                xs.at[c].at[src_v.at[pl.ds(j * CH, CH)]], bufs[p], sems[p])

        def scale(j, p):
            rows = bufs[p]

            def one(i, carry2):
                w = plsc.load_gather(
                    w_v, [jnp.full((16,), j * CH + i, jnp.int32)])
                for k in range(8):
                    rows[i, pl.ds(k * 16, 16)] = (
                        rows[i, pl.ds(k * 16, 16)] * w)
                return carry2

            lax.fori_loop(0, CH, one, 0, unroll=4)

        pltpu.sync_copy(wlh.at[pl.ds(r0, NR)], wl_v)
        for q in range(NR // CH):
            pltpu.sync_copy(xs.at[c, pl.ds(r0 + q * CH, CH)], rows0)

            def init_one(i, carry2):
                w = plsc.load_gather(
                    wl_v, [jnp.full((16,), q * CH + i, jnp.int32)])
                for k in range(8):
                    rows0[i, pl.ds(k * 16, 16)] = (
                        rows0[i, pl.ds(k * 16, 16)] * w)
                return carry2

            lax.fori_loop(0, CH, init_one, 0, unroll=4)
            pltpu.sync_copy(rows0, acc.at[pl.ds(r0 + q * CH, CH)])
        plsc.subcore_barrier()

        npairs = EH // (2 * CH)
        for h in range(NH):
            e0 = s * (NH * EH) + h * EH
            pltpu.sync_copy(srch.at[pl.ds(e0, EH)], src_v)
            pltpu.sync_copy(wh.at[pl.ds(e0, EH)], w_v)
            pltpu.sync_copy(dsth.at[s * NH + h], dst_v)
            gather(0, 0).start()

            def pair(jp, carry):
                c0 = jp * 2
                gather(c0, 0).wait()
                gather(c0 + 1, 1).start()
                scale(c0, 0)
                pltpu.sync_copy(bufs[0], acc.at[dst_v.at[c0]], add=True)
                gather(c0 + 1, 1).wait()

                @pl.when(jp < npairs - 1)
                def _():
                    gather(c0 + 2, 0).start()

                scale(c0 + 1, 1)
                pltpu.sync_copy(bufs[1], acc.at[dst_v.at[c0 + 1]], add=True)
                return carry

            lax.fori_loop(0, npairs, pair, 0)

        plsc.subcore_barrier()
        pltpu.sync_copy(acc.at[pl.ds(r0, NR)], out128.at[c, pl.ds(r0, NR)])

    return pl.kernel(body,
                     out_type=jax.ShapeDtypeStruct((2, NP, 128), jnp.float32),
                     mesh=mesh, scratch_types=scratch,
                     compiler_params=pltpu.CompilerParams(
                         needs_layout_passes=False))


def _make_sc_g_pass():
    mesh = plsc.VectorSubcoreMesh(core_axis_name="c", subcore_axis_name="s")
    scratch = [
        pltpu.VMEM((ETG,), jnp.int32),
        pltpu.VMEM((ETG,), jnp.float32),
        pltpu.VMEM((ETG,), jnp.int32),
        pltpu.VMEM((NP * B,), jnp.float32),
        pltpu.VMEM((NP * B,), jnp.float32),
        pltpu.VMEM((NP,), jnp.float32),
    ]

    def body(inp4h, srch, dst1h, wh, wlh, outg,
             src_v, w_v, dst1_v, inp4, accl, wl_v):
        c = lax.axis_index("c")
        s = lax.axis_index("s")
        t = c * 16 + s
        eg = t * ETG
        pltpu.sync_copy(srch.at[pl.ds(eg, ETG)], src_v)
        pltpu.sync_copy(wh.at[pl.ds(eg, ETG)], w_v)
        pltpu.sync_copy(dst1h.at[pl.ds(eg, ETG)], dst1_v)
        pltpu.sync_copy(inp4h, inp4)
        pltpu.sync_copy(wlh, wl_v)

        def init_group(gi, carry):
            flat = gi * 16 + lax.iota(jnp.int32, 16)
            w = plsc.load_gather(wl_v, [lax.shift_right_logical(flat, 2)])
            accl[pl.ds(gi * 16, 16)] = inp4[pl.ds(gi * 16, 16)] * w
            return carry

        lax.fori_loop(0, NP * B // 16, init_group, 0, unroll=4)

        def group(gi, carry):
            src4 = src_v[pl.ds(gi * 16, 16)] * B
            w16 = w_v[pl.ds(gi * 16, 16)]
            dst4 = dst1_v[pl.ds(gi * 16, 16)] * B
            for b in range(B):
                vals = plsc.load_gather(inp4, [src4 + b]) * w16
                plsc.addupdate_scatter(accl, [dst4 + b], vals)
            return carry

        lax.fori_loop(0, ETG // 16, group, 0, unroll=2)
        pltpu.sync_copy(accl, outg.at[t])

    return pl.kernel(body,
                     out_type=jax.ShapeDtypeStruct((NT, NP * B), jnp.float32),
                     mesh=mesh, scratch_types=scratch,
                     compiler_params=pltpu.CompilerParams(
                         needs_layout_passes=False))


_sc_state = _make_sc_state_pass()
_sc_g = _make_sc_g_pass()

_R = 1024


def _tc1_body(h_ref, g_ref, st_ref, w0_ref, b0_ref, rs_ref, u_ref, g_out):
    g = jnp.sum(g_ref[...], axis=0)
    g_out[...] = g
    w_in = w0_ref[0:1, :]
    w_st = w0_ref[1:65, :]
    for b in range(B):
        c, o = b // 2, (b % 2) * U
        h_b = h_ref[c, :, o:o + U]
        z = (jnp.dot(h_b, w_st, preferred_element_type=jnp.float32)
             + g[:, b:b + 1] * w_in + b0_ref[:])
        v = jax.nn.sigmoid(z)
        r, u = v[:, :U], v[:, U:]
        rs_ref[c, :, o:o + U] = r * st_ref[c, :, o:o + U]
        u_ref[c, :, o:o + U] = u


def _tc2_body(h_ref, g_ref, st_ref, u_ref, w1_ref, b1_ref, ns_ref):
    g = g_ref[...]
    w_in = w1_ref[0:1, :]
    w_st = w1_ref[1:65, :]
    for b in range(B):
        c, o = b // 2, (b % 2) * U
        h_b = h_ref[c, :, o:o + U]
        cand = jnp.tanh(jnp.dot(h_b, w_st, preferred_element_type=jnp.float32)
                        + g[:, b:b + 1] * w_in + b1_ref[:])
        u = u_ref[c, :, o:o + U]
        ns_ref[b, :, :] = u * st_ref[c, :, o:o + U] + (1.0 - u) * cand


def _stacked_spec():
    return pl.BlockSpec((2, _R, 128), lambda i: (0, i, 0))


def _tc1(h1, gp, xs, w0, b0):
    return pl.pallas_call(
        _tc1_body,
        grid=(NP // _R,),
        in_specs=[
            _stacked_spec(),
            pl.BlockSpec((NT, _R, B), lambda i: (0, i, 0)),
            _stacked_spec(),
            pl.BlockSpec((65, 128), lambda i: (0, 0)),
            pl.BlockSpec((1, 128), lambda i: (0, 0)),
        ],
        out_specs=[_stacked_spec(), _stacked_spec(),
                   pl.BlockSpec((_R, B), lambda i: (i, 0))],
        out_shape=[jax.ShapeDtypeStruct((2, NP, 128), jnp.float32),
                   jax.ShapeDtypeStruct((2, NP, 128), jnp.float32),
                   jax.ShapeDtypeStruct((NP, B), jnp.float32)],
    )(h1, gp, xs, w0, b0)


def _tc2(h2, g, xs, u, w1, b1):
    return pl.pallas_call(
        _tc2_body,
        grid=(NP // _R,),
        in_specs=[
            _stacked_spec(),
            pl.BlockSpec((_R, B), lambda i: (i, 0)),
            _stacked_spec(),
            _stacked_spec(),
            pl.BlockSpec((65, 64), lambda i: (0, 0)),
            pl.BlockSpec((1, 64), lambda i: (0, 0)),
        ],
        out_specs=pl.BlockSpec((B, _R, U), lambda i: (0, i, 0)),
        out_shape=jax.ShapeDtypeStruct((B, NP, U), jnp.float32),
    )(h2, g, xs, u, w1, b1)


@jax.jit
def kernel(inputs, state, edge_src, edge_dst, edge_weight,
           weights_0, bias_0, weights_1, bias_1):
    xs = (state.reshape(2, 2, N, U).transpose(0, 2, 1, 3)
          .reshape(2, N, 2 * U))
    xs = jnp.pad(xs, ((0, 0), (0, NP - N), (0, 0)))
    inp4 = jnp.pad(inputs.reshape(B, N).T, ((0, NP - N), (0, 0))).reshape(-1)
    wl = jnp.pad(edge_weight[E0:], (0, NP - N))
    pad = EP - E0
    src_p = jnp.concatenate([edge_src[:E0], jnp.zeros((pad,), jnp.int32)])
    dst_p = jnp.concatenate([edge_dst[:E0], jnp.zeros((pad,), jnp.int32)])
    w_p = jnp.concatenate([edge_weight[:E0], jnp.zeros((pad,), jnp.float32)])
    dsth = dst_p.reshape(16 * NH, EH // CH, CH)

    gp = _sc_g(inp4, src_p, dst_p, w_p, wl)
    h1 = _sc_state(xs, src_p, dsth, w_p, wl)
    gp = gp.reshape(NT, NP, B)
    rs, u, g = _tc1(h1, gp, xs, weights_0, bias_0.reshape(1, -1))
    h2 = _sc_state(rs, src_p, dsth, w_p, wl)
    ns = _tc2(h2, g, xs, u, weights_1, bias_1.reshape(1, -1))
    return ns[:, :N, :].reshape(B, N * U)


# speedup vs baseline: 1.2269x; 1.2269x over previous
import jax
import jax.numpy as jnp
from jax import lax
from jax.experimental import pallas as pl
from jax.experimental.pallas import tpu as pltpu
from jax.experimental.pallas import tpu_sc as plsc

N = 10000
B = 4
U = 64
E = 160000 + N
EP = 172032
CH = 128
EH = 3584
NH = EP // (16 * EH)
NP = 10240
NR = NP // 16
ETG = EP // 32
NT = 32


def _make_sc_state_pass():
    mesh = plsc.VectorSubcoreMesh(core_axis_name="c", subcore_axis_name="s")
    scratch = [
        pltpu.VMEM_SHARED((NP, 128), jnp.float32),
        pltpu.VMEM((EH,), jnp.int32),
        pltpu.VMEM((EH,), jnp.float32),
        pltpu.VMEM((EH // CH, CH), jnp.int32),
        pltpu.VMEM((CH, 128), jnp.float32),
        pltpu.VMEM((CH, 128), jnp.float32),
        pltpu.SemaphoreType.DMA,
        pltpu.SemaphoreType.DMA,
    ]

    def body(xs, srch, dsth, wh, z128,
             out128, acc, src_v, w_v, dst_v, rows0, rows1, sem0, sem1):
        c = lax.axis_index("c")
        s = lax.axis_index("s")
        r0 = s * NR
        bufs = (rows0, rows1)
        sems = (sem0, sem1)

        def gather(j, p):
            return pltpu.make_async_copy(
                xs.at[c].at[src_v.at[pl.ds(j * CH, CH)]], bufs[p], sems[p])

        def scale(j, p):
            rows = bufs[p]

            def one(i, carry2):
                w = plsc.load_gather(
                    w_v, [jnp.full((16,), j * CH + i, jnp.int32)])
                for k in range(8):
                    rows[i, pl.ds(k * 16, 16)] = (
                        rows[i, pl.ds(k * 16, 16)] * w)
                return carry2

            lax.fori_loop(0, CH, one, 0, unroll=4)

        pltpu.sync_copy(z128.at[pl.ds(r0, NR)], acc.at[pl.ds(r0, NR)])
        plsc.subcore_barrier()

        npairs = EH // (2 * CH)
        for h in range(NH):
            e0 = s * (NH * EH) + h * EH
            pltpu.sync_copy(srch.at[pl.ds(e0, EH)], src_v)
            pltpu.sync_copy(wh.at[pl.ds(e0, EH)], w_v)
            pltpu.sync_copy(dsth.at[s * NH + h], dst_v)
            gather(0, 0).start()

            def pair(jp, carry):
                c0 = jp * 2
                gather(c0, 0).wait()
                gather(c0 + 1, 1).start()
                scale(c0, 0)
                pltpu.sync_copy(bufs[0], acc.at[dst_v.at[c0]], add=True)
                gather(c0 + 1, 1).wait()

                @pl.when(jp < npairs - 1)
                def _():
                    gather(c0 + 2, 0).start()

                scale(c0 + 1, 1)
                pltpu.sync_copy(bufs[1], acc.at[dst_v.at[c0 + 1]], add=True)
                return carry

            lax.fori_loop(0, npairs, pair, 0)

        plsc.subcore_barrier()
        pltpu.sync_copy(acc.at[pl.ds(r0, NR)], out128.at[c, pl.ds(r0, NR)])

    return pl.kernel(body,
                     out_type=jax.ShapeDtypeStruct((2, NP, 128), jnp.float32),
                     mesh=mesh, scratch_types=scratch,
                     compiler_params=pltpu.CompilerParams(
                         needs_layout_passes=False))


def _make_sc_g_pass():
    mesh = plsc.VectorSubcoreMesh(core_axis_name="c", subcore_axis_name="s")
    scratch = [
        pltpu.VMEM((ETG,), jnp.int32),
        pltpu.VMEM((ETG,), jnp.float32),
        pltpu.VMEM((ETG,), jnp.int32),
        pltpu.VMEM((NP * B,), jnp.float32),
        pltpu.VMEM((NP * B,), jnp.float32),
    ]

    def body(inp4h, srch, dst1h, wh, z4, outg,
             src_v, w_v, dst1_v, inp4, accl):
        c = lax.axis_index("c")
        s = lax.axis_index("s")
        t = c * 16 + s
        eg = t * ETG
        pltpu.sync_copy(srch.at[pl.ds(eg, ETG)], src_v)
        pltpu.sync_copy(wh.at[pl.ds(eg, ETG)], w_v)
        pltpu.sync_copy(dst1h.at[pl.ds(eg, ETG)], dst1_v)
        pltpu.sync_copy(inp4h, inp4)
        pltpu.sync_copy(z4, accl)

        def group(gi, carry):
            src4 = src_v[pl.ds(gi * 16, 16)] * B
            w16 = w_v[pl.ds(gi * 16, 16)]
            dst4 = dst1_v[pl.ds(gi * 16, 16)] * B
            for b in range(B):
                vals = plsc.load_gather(inp4, [src4 + b]) * w16
                plsc.addupdate_scatter(accl, [dst4 + b], vals)
            return carry

        lax.fori_loop(0, ETG // 16, group, 0, unroll=2)
        pltpu.sync_copy(accl, outg.at[t])

    return pl.kernel(body,
                     out_type=jax.ShapeDtypeStruct((NT, NP * B), jnp.float32),
                     mesh=mesh, scratch_types=scratch,
                     compiler_params=pltpu.CompilerParams(
                         needs_layout_passes=False))


_sc_state = _make_sc_state_pass()
_sc_g = _make_sc_g_pass()

_R = 1024


def _tc1_body(h_ref, g_ref, st_ref, w0_ref, b0_ref, rs_ref, u_ref, g_out):
    g = jnp.sum(g_ref[...], axis=0)
    g_out[...] = g
    w_in = w0_ref[0:1, :]
    w_st = w0_ref[1:65, :]
    for b in range(B):
        c, o = b // 2, (b % 2) * U
        h_b = h_ref[c, :, o:o + U]
        z = (jnp.dot(h_b, w_st, preferred_element_type=jnp.float32)
             + g[:, b:b + 1] * w_in + b0_ref[:])
        v = jax.nn.sigmoid(z)
        r, u = v[:, :U], v[:, U:]
        rs_ref[c, :, o:o + U] = r * st_ref[c, :, o:o + U]
        u_ref[c, :, o:o + U] = u


def _tc2_body(h_ref, g_ref, st_ref, u_ref, w1_ref, b1_ref, ns_ref):
    g = g_ref[...]
    w_in = w1_ref[0:1, :]
    w_st = w1_ref[1:65, :]
    for b in range(B):
        c, o = b // 2, (b % 2) * U
        h_b = h_ref[c, :, o:o + U]
        cand = jnp.tanh(jnp.dot(h_b, w_st, preferred_element_type=jnp.float32)
                        + g[:, b:b + 1] * w_in + b1_ref[:])
        u = u_ref[c, :, o:o + U]
        ns_ref[b, :, :] = u * st_ref[c, :, o:o + U] + (1.0 - u) * cand


def _stacked_spec():
    return pl.BlockSpec((2, _R, 128), lambda i: (0, i, 0))


def _tc1(h1, gp, xs, w0, b0):
    return pl.pallas_call(
        _tc1_body,
        grid=(NP // _R,),
        in_specs=[
            _stacked_spec(),
            pl.BlockSpec((NT, _R, B), lambda i: (0, i, 0)),
            _stacked_spec(),
            pl.BlockSpec((65, 128), lambda i: (0, 0)),
            pl.BlockSpec((1, 128), lambda i: (0, 0)),
        ],
        out_specs=[_stacked_spec(), _stacked_spec(),
                   pl.BlockSpec((_R, B), lambda i: (i, 0))],
        out_shape=[jax.ShapeDtypeStruct((2, NP, 128), jnp.float32),
                   jax.ShapeDtypeStruct((2, NP, 128), jnp.float32),
                   jax.ShapeDtypeStruct((NP, B), jnp.float32)],
    )(h1, gp, xs, w0, b0)


def _tc2(h2, g, xs, u, w1, b1):
    return pl.pallas_call(
        _tc2_body,
        grid=(NP // _R,),
        in_specs=[
            _stacked_spec(),
            pl.BlockSpec((_R, B), lambda i: (i, 0)),
            _stacked_spec(),
            _stacked_spec(),
            pl.BlockSpec((65, 64), lambda i: (0, 0)),
            pl.BlockSpec((1, 64), lambda i: (0, 0)),
        ],
        out_specs=pl.BlockSpec((B, _R, U), lambda i: (0, i, 0)),
        out_shape=jax.ShapeDtypeStruct((B, NP, U), jnp.float32),
    )(h2, g, xs, u, w1, b1)


@jax.jit
def kernel(inputs, state, edge_src, edge_dst, edge_weight,
           weights_0, bias_0, weights_1, bias_1):
    xs = (state.reshape(2, 2, N, U).transpose(0, 2, 1, 3)
          .reshape(2, N, 2 * U))
    xs = jnp.pad(xs, ((0, 0), (0, NP - N), (0, 0)))
    inp4 = jnp.pad(inputs.reshape(B, N).T, ((0, NP - N), (0, 0))).reshape(-1)
    pad = EP - E
    src_p = jnp.concatenate([edge_src, jnp.zeros((pad,), jnp.int32)])
    dst_p = jnp.concatenate([edge_dst, jnp.zeros((pad,), jnp.int32)])
    w_p = jnp.concatenate([edge_weight, jnp.zeros((pad,), jnp.float32)])
    dsth = dst_p.reshape(16 * NH, EH // CH, CH)
    z128 = jnp.zeros((NP, 128), jnp.float32)
    z4 = jnp.zeros((NP * B,), jnp.float32)

    gp = _sc_g(inp4, src_p, dst_p, w_p, z4)
    h1 = _sc_state(xs, src_p, dsth, w_p, z128)
    gp = gp.reshape(NT, NP, B)
    rs, u, g = _tc1(h1, gp, xs, weights_0, bias_0.reshape(1, -1))
    h2 = _sc_state(rs, src_p, dsth, w_p, z128)
    ns = _tc2(h2, g, xs, u, weights_1, bias_1.reshape(1, -1))
    return ns[:, :N, :].reshape(B, N * U)


# speedup vs baseline: 1.2886x; 1.0502x over previous
import jax
import jax.numpy as jnp
from jax import lax
from jax.experimental import pallas as pl
from jax.experimental.pallas import tpu as pltpu
from jax.experimental.pallas import tpu_sc as plsc

N = 10000
B = 4
U = 64
E = 160000 + N
EP = 172032
CH = 128
EH = 3584
NH = EP // (16 * EH)
NP = 10240
NR = NP // 16
ETG = EP // 32
NT = 32


def _make_sc_state_pass():
    mesh = plsc.VectorSubcoreMesh(core_axis_name="c", subcore_axis_name="s")
    scratch = [
        pltpu.VMEM_SHARED((NP, 128), jnp.float32),
        pltpu.VMEM((EH,), jnp.int32),
        pltpu.VMEM((EH,), jnp.float32),
        pltpu.VMEM((EH // CH, CH), jnp.int32),
        pltpu.VMEM((CH, 128), jnp.float32),
        pltpu.VMEM((CH, 128), jnp.float32),
        pltpu.SemaphoreType.DMA,
        pltpu.SemaphoreType.DMA,
    ]

    def body(xs, srch, dsth, wh, z128,
             out128, acc, src_v, w_v, dst_v, rows0, rows1, sem0, sem1):
        c = lax.axis_index("c")
        s = lax.axis_index("s")
        r0 = s * NR
        bufs = (rows0, rows1)
        sems = (sem0, sem1)

        def gather(j, p):
            return pltpu.make_async_copy(
                xs.at[c].at[src_v.at[pl.ds(j * CH, CH)]], bufs[p], sems[p])

        def scale(j, p):
            rows = bufs[p]

            @plsc.parallel_loop(0, CH, unroll=4)
            def one(i):
                w = plsc.load_gather(
                    w_v, [jnp.full((16,), j * CH + i, jnp.int32)])
                for k in range(8):
                    rows[i, pl.ds(k * 16, 16)] = (
                        rows[i, pl.ds(k * 16, 16)] * w)

        pltpu.sync_copy(z128.at[pl.ds(r0, NR)], acc.at[pl.ds(r0, NR)])
        plsc.subcore_barrier()

        npairs = EH // (2 * CH)
        for h in range(NH):
            e0 = s * (NH * EH) + h * EH
            pltpu.sync_copy(srch.at[pl.ds(e0, EH)], src_v)
            pltpu.sync_copy(wh.at[pl.ds(e0, EH)], w_v)
            pltpu.sync_copy(dsth.at[s * NH + h], dst_v)
            gather(0, 0).start()

            def pair(jp, carry):
                c0 = jp * 2
                gather(c0, 0).wait()
                gather(c0 + 1, 1).start()
                scale(c0, 0)
                pltpu.sync_copy(bufs[0], acc.at[dst_v.at[c0]], add=True)
                gather(c0 + 1, 1).wait()

                @pl.when(jp < npairs - 1)
                def _():
                    gather(c0 + 2, 0).start()

                scale(c0 + 1, 1)
                pltpu.sync_copy(bufs[1], acc.at[dst_v.at[c0 + 1]], add=True)
                return carry

            lax.fori_loop(0, npairs, pair, 0)

        plsc.subcore_barrier()
        pltpu.sync_copy(acc.at[pl.ds(r0, NR)], out128.at[c, pl.ds(r0, NR)])

    return pl.kernel(body,
                     out_type=jax.ShapeDtypeStruct((2, NP, 128), jnp.float32),
                     mesh=mesh, scratch_types=scratch,
                     compiler_params=pltpu.CompilerParams(
                         needs_layout_passes=False))


def _make_sc_g_pass():
    mesh = plsc.VectorSubcoreMesh(core_axis_name="c", subcore_axis_name="s")
    scratch = [
        pltpu.VMEM((ETG,), jnp.int32),
        pltpu.VMEM((ETG,), jnp.float32),
        pltpu.VMEM((ETG,), jnp.int32),
        pltpu.VMEM((NP * B,), jnp.float32),
        pltpu.VMEM((NP * B,), jnp.float32),
    ]

    def body(inp4h, srch, dst1h, wh, z4, outg,
             src_v, w_v, dst1_v, inp4, accl):
        c = lax.axis_index("c")
        s = lax.axis_index("s")
        t = c * 16 + s
        eg = t * ETG
        pltpu.sync_copy(srch.at[pl.ds(eg, ETG)], src_v)
        pltpu.sync_copy(wh.at[pl.ds(eg, ETG)], w_v)
        pltpu.sync_copy(dst1h.at[pl.ds(eg, ETG)], dst1_v)
        pltpu.sync_copy(inp4h, inp4)
        pltpu.sync_copy(z4, accl)

        def group(gi, carry):
            src4 = src_v[pl.ds(gi * 16, 16)] * B
            w16 = w_v[pl.ds(gi * 16, 16)]
            dst4 = dst1_v[pl.ds(gi * 16, 16)] * B
            for b in range(B):
                vals = plsc.load_gather(inp4, [src4 + b]) * w16
                plsc.addupdate_scatter(accl, [dst4 + b], vals)
            return carry

        lax.fori_loop(0, ETG // 16, group, 0, unroll=2)
        pltpu.sync_copy(accl, outg.at[t])

    return pl.kernel(body,
                     out_type=jax.ShapeDtypeStruct((NT, NP * B), jnp.float32),
                     mesh=mesh, scratch_types=scratch,
                     compiler_params=pltpu.CompilerParams(
                         needs_layout_passes=False))


_sc_state = _make_sc_state_pass()
_sc_g = _make_sc_g_pass()

_R = 1024


def _tc1_body(h_ref, g_ref, st_ref, w0_ref, b0_ref, rs_ref, u_ref, g_out):
    g = jnp.sum(g_ref[...], axis=0)
    g_out[...] = g
    w_in = w0_ref[0:1, :]
    w_st = w0_ref[1:65, :]
    for b in range(B):
        c, o = b // 2, (b % 2) * U
        h_b = h_ref[c, :, o:o + U]
        z = (jnp.dot(h_b, w_st, preferred_element_type=jnp.float32)
             + g[:, b:b + 1] * w_in + b0_ref[:])
        v = jax.nn.sigmoid(z)
        r, u = v[:, :U], v[:, U:]
        rs_ref[c, :, o:o + U] = r * st_ref[c, :, o:o + U]
        u_ref[c, :, o:o + U] = u


def _tc2_body(h_ref, g_ref, st_ref, u_ref, w1_ref, b1_ref, ns_ref):
    g = g_ref[...]
    w_in = w1_ref[0:1, :]
    w_st = w1_ref[1:65, :]
    for b in range(B):
        c, o = b // 2, (b % 2) * U
        h_b = h_ref[c, :, o:o + U]
        cand = jnp.tanh(jnp.dot(h_b, w_st, preferred_element_type=jnp.float32)
                        + g[:, b:b + 1] * w_in + b1_ref[:])
        u = u_ref[c, :, o:o + U]
        ns_ref[b, :, :] = u * st_ref[c, :, o:o + U] + (1.0 - u) * cand


def _stacked_spec():
    return pl.BlockSpec((2, _R, 128), lambda i: (0, i, 0))


def _tc1(h1, gp, xs, w0, b0):
    return pl.pallas_call(
        _tc1_body,
        grid=(NP // _R,),
        in_specs=[
            _stacked_spec(),
            pl.BlockSpec((NT, _R, B), lambda i: (0, i, 0)),
            _stacked_spec(),
            pl.BlockSpec((65, 128), lambda i: (0, 0)),
            pl.BlockSpec((1, 128), lambda i: (0, 0)),
        ],
        out_specs=[_stacked_spec(), _stacked_spec(),
                   pl.BlockSpec((_R, B), lambda i: (i, 0))],
        out_shape=[jax.ShapeDtypeStruct((2, NP, 128), jnp.float32),
                   jax.ShapeDtypeStruct((2, NP, 128), jnp.float32),
                   jax.ShapeDtypeStruct((NP, B), jnp.float32)],
    )(h1, gp, xs, w0, b0)


def _tc2(h2, g, xs, u, w1, b1):
    return pl.pallas_call(
        _tc2_body,
        grid=(NP // _R,),
        in_specs=[
            _stacked_spec(),
            pl.BlockSpec((_R, B), lambda i: (i, 0)),
            _stacked_spec(),
            _stacked_spec(),
            pl.BlockSpec((65, 64), lambda i: (0, 0)),
            pl.BlockSpec((1, 64), lambda i: (0, 0)),
        ],
        out_specs=pl.BlockSpec((B, _R, U), lambda i: (0, i, 0)),
        out_shape=jax.ShapeDtypeStruct((B, NP, U), jnp.float32),
    )(h2, g, xs, u, w1, b1)


@jax.jit
def kernel(inputs, state, edge_src, edge_dst, edge_weight,
           weights_0, bias_0, weights_1, bias_1):
    xs = (state.reshape(2, 2, N, U).transpose(0, 2, 1, 3)
          .reshape(2, N, 2 * U))
    xs = jnp.pad(xs, ((0, 0), (0, NP - N), (0, 0)))
    inp4 = jnp.pad(inputs.reshape(B, N).T, ((0, NP - N), (0, 0))).reshape(-1)
    pad = EP - E
    src_p = jnp.concatenate([edge_src, jnp.zeros((pad,), jnp.int32)])
    dst_p = jnp.concatenate([edge_dst, jnp.zeros((pad,), jnp.int32)])
    w_p = jnp.concatenate([edge_weight, jnp.zeros((pad,), jnp.float32)])
    dsth = dst_p.reshape(16 * NH, EH // CH, CH)
    z128 = jnp.zeros((NP, 128), jnp.float32)
    z4 = jnp.zeros((NP * B,), jnp.float32)

    gp = _sc_g(inp4, src_p, dst_p, w_p, z4)
    h1 = _sc_state(xs, src_p, dsth, w_p, z128)
    gp = gp.reshape(NT, NP, B)
    rs, u, g = _tc1(h1, gp, xs, weights_0, bias_0.reshape(1, -1))
    h2 = _sc_state(rs, src_p, dsth, w_p, z128)
    ns = _tc2(h2, g, xs, u, weights_1, bias_1.reshape(1, -1))
    return ns[:, :N, :].reshape(B, N * U)


# speedup vs baseline: 1.2979x; 1.0073x over previous
import jax
import jax.numpy as jnp
from jax import lax
from jax.experimental import pallas as pl
from jax.experimental.pallas import tpu as pltpu
from jax.experimental.pallas import tpu_sc as plsc

N = 10000
B = 4
U = 64
E = 160000 + N
EP = 172032
CH = 128
EH = 3584
NH = EP // (16 * EH)
NP = 10240
NR = NP // 16
ETG = EP // 32
NT = 32


def _make_sc_state_pass():
    mesh = plsc.VectorSubcoreMesh(core_axis_name="c", subcore_axis_name="s")
    scratch = [
        pltpu.VMEM_SHARED((NP, 128), jnp.float32),
        pltpu.VMEM((EH,), jnp.int32),
        pltpu.VMEM((EH,), jnp.float32),
        pltpu.VMEM((EH // CH, CH), jnp.int32),
        pltpu.VMEM((CH, 128), jnp.float32),
        pltpu.VMEM((CH, 128), jnp.float32),
        pltpu.SemaphoreType.DMA,
        pltpu.SemaphoreType.DMA,
    ]

    def body(xs, srch, dsth, wh, z128,
             out128, acc, src_v, w_v, dst_v, rows0, rows1, sem0, sem1):
        c = lax.axis_index("c")
        s = lax.axis_index("s")
        r0 = s * NR
        bufs = (rows0, rows1)
        sems = (sem0, sem1)

        def gather(j, p):
            return pltpu.make_async_copy(
                xs.at[c].at[src_v.at[pl.ds(j * CH, CH)]], bufs[p], sems[p])

        def scale(j, p):
            rows = bufs[p]

            @plsc.parallel_loop(0, CH, unroll=4)
            def one(i):
                w = plsc.load_gather(
                    w_v, [jnp.full((16,), j * CH + i, jnp.int32)])
                for k in range(8):
                    rows[i, pl.ds(k * 16, 16)] = (
                        rows[i, pl.ds(k * 16, 16)] * w)

        pltpu.sync_copy(z128.at[pl.ds(r0, NR)], acc.at[pl.ds(r0, NR)])
        plsc.subcore_barrier()

        npairs = EH // (2 * CH)
        for h in range(NH):
            e0 = s * (NH * EH) + h * EH
            pltpu.sync_copy(srch.at[pl.ds(e0, EH)], src_v)
            gather(0, 0).start()
            pltpu.sync_copy(wh.at[pl.ds(e0, EH)], w_v)
            pltpu.sync_copy(dsth.at[s * NH + h], dst_v)

            def pair(jp, carry):
                c0 = jp * 2
                gather(c0, 0).wait()
                gather(c0 + 1, 1).start()
                scale(c0, 0)
                pltpu.sync_copy(bufs[0], acc.at[dst_v.at[c0]], add=True)
                gather(c0 + 1, 1).wait()

                @pl.when(jp < npairs - 1)
                def _():
                    gather(c0 + 2, 0).start()

                scale(c0 + 1, 1)
                pltpu.sync_copy(bufs[1], acc.at[dst_v.at[c0 + 1]], add=True)
                return carry

            lax.fori_loop(0, npairs, pair, 0)

        plsc.subcore_barrier()
        pltpu.sync_copy(acc.at[pl.ds(r0, NR)], out128.at[c, pl.ds(r0, NR)])

    return pl.kernel(body,
                     out_type=jax.ShapeDtypeStruct((2, NP, 128), jnp.float32),
                     mesh=mesh, scratch_types=scratch,
                     compiler_params=pltpu.CompilerParams(
                         needs_layout_passes=False))


def _make_sc_g_pass():
    mesh = plsc.VectorSubcoreMesh(core_axis_name="c", subcore_axis_name="s")
    scratch = [
        pltpu.VMEM((ETG,), jnp.int32),
        pltpu.VMEM((ETG,), jnp.float32),
        pltpu.VMEM((ETG,), jnp.int32),
        pltpu.VMEM((NP * B,), jnp.float32),
        pltpu.VMEM((NP * B,), jnp.float32),
    ]

    def body(inp4h, srch, dst1h, wh, z4, outg,
             src_v, w_v, dst1_v, inp4, accl):
        c = lax.axis_index("c")
        s = lax.axis_index("s")
        t = c * 16 + s
        eg = t * ETG
        pltpu.sync_copy(srch.at[pl.ds(eg, ETG)], src_v)
        pltpu.sync_copy(wh.at[pl.ds(eg, ETG)], w_v)
        pltpu.sync_copy(dst1h.at[pl.ds(eg, ETG)], dst1_v)
        pltpu.sync_copy(inp4h, inp4)
        pltpu.sync_copy(z4, accl)

        def group(gi, carry):
            src4 = src_v[pl.ds(gi * 16, 16)] * B
            w16 = w_v[pl.ds(gi * 16, 16)]
            dst4 = dst1_v[pl.ds(gi * 16, 16)] * B
            for b in range(B):
                vals = plsc.load_gather(inp4, [src4 + b]) * w16
                plsc.addupdate_scatter(accl, [dst4 + b], vals)
            return carry

        lax.fori_loop(0, ETG // 16, group, 0, unroll=2)
        pltpu.sync_copy(accl, outg.at[t])

    return pl.kernel(body,
                     out_type=jax.ShapeDtypeStruct((NT, NP * B), jnp.float32),
                     mesh=mesh, scratch_types=scratch,
                     compiler_params=pltpu.CompilerParams(
                         needs_layout_passes=False))


_sc_state = _make_sc_state_pass()
_sc_g = _make_sc_g_pass()

_R = 1024


def _tc1_body(h_ref, g_ref, st_ref, w0_ref, b0_ref, rs_ref, u_ref, g_out):
    g = jnp.sum(g_ref[...], axis=0)
    g_out[...] = g
    w_in = w0_ref[0:1, :]
    w_st = w0_ref[1:65, :]
    for b in range(B):
        c, o = b // 2, (b % 2) * U
        h_b = h_ref[c, :, o:o + U]
        z = (jnp.dot(h_b, w_st, preferred_element_type=jnp.float32)
             + g[:, b:b + 1] * w_in + b0_ref[:])
        v = jax.nn.sigmoid(z)
        r, u = v[:, :U], v[:, U:]
        rs_ref[c, :, o:o + U] = r * st_ref[c, :, o:o + U]
        u_ref[c, :, o:o + U] = u


def _tc2_body(h_ref, g_ref, st_ref, u_ref, w1_ref, b1_ref, ns_ref):
    g = g_ref[...]
    w_in = w1_ref[0:1, :]
    w_st = w1_ref[1:65, :]
    for b in range(B):
        c, o = b // 2, (b % 2) * U
        h_b = h_ref[c, :, o:o + U]
        cand = jnp.tanh(jnp.dot(h_b, w_st, preferred_element_type=jnp.float32)
                        + g[:, b:b + 1] * w_in + b1_ref[:])
        u = u_ref[c, :, o:o + U]
        ns_ref[b, :, :] = u * st_ref[c, :, o:o + U] + (1.0 - u) * cand


def _stacked_spec():
    return pl.BlockSpec((2, _R, 128), lambda i: (0, i, 0))


def _tc1(h1, gp, xs, w0, b0):
    return pl.pallas_call(
        _tc1_body,
        grid=(NP // _R,),
        in_specs=[
            _stacked_spec(),
            pl.BlockSpec((NT, _R, B), lambda i: (0, i, 0)),
            _stacked_spec(),
            pl.BlockSpec((65, 128), lambda i: (0, 0)),
            pl.BlockSpec((1, 128), lambda i: (0, 0)),
        ],
        out_specs=[_stacked_spec(), _stacked_spec(),
                   pl.BlockSpec((_R, B), lambda i: (i, 0))],
        out_shape=[jax.ShapeDtypeStruct((2, NP, 128), jnp.float32),
                   jax.ShapeDtypeStruct((2, NP, 128), jnp.float32),
                   jax.ShapeDtypeStruct((NP, B), jnp.float32)],
    )(h1, gp, xs, w0, b0)


def _tc2(h2, g, xs, u, w1, b1):
    return pl.pallas_call(
        _tc2_body,
        grid=(NP // _R,),
        in_specs=[
            _stacked_spec(),
            pl.BlockSpec((_R, B), lambda i: (i, 0)),
            _stacked_spec(),
            _stacked_spec(),
            pl.BlockSpec((65, 64), lambda i: (0, 0)),
            pl.BlockSpec((1, 64), lambda i: (0, 0)),
        ],
        out_specs=pl.BlockSpec((B, _R, U), lambda i: (0, i, 0)),
        out_shape=jax.ShapeDtypeStruct((B, NP, U), jnp.float32),
    )(h2, g, xs, u, w1, b1)


@jax.jit
def kernel(inputs, state, edge_src, edge_dst, edge_weight,
           weights_0, bias_0, weights_1, bias_1):
    xs = (state.reshape(2, 2, N, U).transpose(0, 2, 1, 3)
          .reshape(2, N, 2 * U))
    xs = jnp.pad(xs, ((0, 0), (0, NP - N), (0, 0)))
    inp4 = jnp.pad(inputs.reshape(B, N).T, ((0, NP - N), (0, 0))).reshape(-1)
    pad = EP - E
    src_p = jnp.concatenate([edge_src, jnp.zeros((pad,), jnp.int32)])
    dst_p = jnp.concatenate([edge_dst, jnp.zeros((pad,), jnp.int32)])
    w_p = jnp.concatenate([edge_weight, jnp.zeros((pad,), jnp.float32)])
    dsth = dst_p.reshape(16 * NH, EH // CH, CH)
    z128 = jnp.zeros((NP, 128), jnp.float32)
    z4 = jnp.zeros((NP * B,), jnp.float32)

    gp = _sc_g(inp4, src_p, dst_p, w_p, z4)
    h1 = _sc_state(xs, src_p, dsth, w_p, z128)
    gp = gp.reshape(NT, NP, B)
    rs, u, g = _tc1(h1, gp, xs, weights_0, bias_0.reshape(1, -1))
    h2 = _sc_state(rs, src_p, dsth, w_p, z128)
    ns = _tc2(h2, g, xs, u, weights_1, bias_1.reshape(1, -1))
    return ns[:, :N, :].reshape(B, N * U)
